# Initial kernel scaffold; baseline (speedup 1.0000x reference)
import math
import functools

import jax
import jax.numpy as jnp
from jax.experimental import pallas as pl
from jax.experimental.pallas import tpu as pltpu

D = 768
H = 12
HD = 64
Q = 256
K = 4096
KTU = max(32, min(int(math.ceil(0.1 * K)), K))
SCALE = HD ** -0.5

_ONE_BITS = 0x3F800000


def _seg_matrix():
    lane = jax.lax.broadcasted_iota(jnp.int32, (D, H), 0)
    head = jax.lax.broadcasted_iota(jnp.int32, (D, H), 1)
    return (lane // HD == head).astype(jnp.float32)


def _ln_norm_proj(x, g, b, W, bias):
    m = jnp.mean(x, axis=-1, keepdims=True)
    v = jnp.mean((x - m) ** 2, axis=-1, keepdims=True)
    xn = (x - m) * jax.lax.rsqrt(v + 1e-5) * g + b
    return jnp.dot(xn, W, preferred_element_type=jnp.float32) + bias


def _headwise_l2norm(x, seg):
    ssum = jnp.dot(x * x, seg, preferred_element_type=jnp.float32)
    nrm = jnp.maximum(jnp.sqrt(ssum), 1e-12)
    inv_full = jnp.dot(1.0 / nrm, seg.T, preferred_element_type=jnp.float32)
    return x * inv_full


def _stage_a_kernel(query_ref, qpos_ref, key_ref, kpos_ref,
                    q_ln_g_ref, q_ln_b_ref, q_W_ref, q_b_ref,
                    k_ln_g_ref, k_ln_b_ref, k_W_ref, k_b_ref,
                    q4n_ref, qp_ref, k4n_ref, v_ref):
    seg = _seg_matrix()
    q = query_ref[...] + qpos_ref[...]
    qp = _ln_norm_proj(q, q_ln_g_ref[...], q_ln_b_ref[...],
                       q_W_ref[...], q_b_ref[...])
    qp_ref[...] = qp
    q4n_ref[...] = _headwise_l2norm(qp, seg)
    kk = key_ref[...] + kpos_ref[...]
    v_ref[...] = kk
    kp = _ln_norm_proj(kk, k_ln_g_ref[...], k_ln_b_ref[...],
                       k_W_ref[...], k_b_ref[...])
    k4n_ref[...] = _headwise_l2norm(kp, seg)


def _encode(x):
    i = jax.lax.bitcast_convert_type(x, jnp.int32)
    return i ^ jax.lax.shift_right_logical(
        jax.lax.shift_right_arithmetic(i, 31), 1)


def _stage_b_kernel(q_ref, k_ref, v_ref, out_ref):
    q = q_ref[0]
    k = k_ref[0]
    v = v_ref[0]
    sim = jax.lax.dot_general(q, k, (((1,), (1,)), ((), ())),
                              preferred_element_type=jnp.float32)
    enc = _encode(sim)

    def body(_, carry):
        lo, hi = carry
        mid = lo + jax.lax.shift_right_arithmetic(hi - lo, 1)
        cnt = jnp.sum((enc >= mid).astype(jnp.int32), axis=-1, keepdims=True)
        ge = cnt >= KTU
        return jnp.where(ge, mid, lo), jnp.where(ge, hi, mid)

    lo0 = jnp.full((Q, 1), -(_ONE_BITS + 1), jnp.int32)
    hi0 = jnp.full((Q, 1), _ONE_BITS + 1, jnp.int32)
    lo, _ = jax.lax.fori_loop(0, 31, body, (lo0, hi0))

    p = jnp.where(enc >= lo, jnp.exp(sim * SCALE), 0.0)
    attn = p / jnp.sum(p, axis=-1, keepdims=True)
    out_ref[0] = jnp.dot(attn, v, preferred_element_type=jnp.float32)


def _stage_c_kernel(merge_ref, qp_ref, residual_ref,
                    p_W_ref, p_b_ref, f_W_ref, f_b_ref, alpha_ref, out_ref):
    merge = merge_ref[...]
    inter = jnp.dot(merge * qp_ref[...], p_W_ref[...],
                    preferred_element_type=jnp.float32) + p_b_ref[...]
    n2 = jnp.sum(inter * inter, axis=0, keepdims=True)
    nrm = jnp.maximum(jnp.sqrt(n2), 1e-12)
    out = inter / nrm * alpha_ref[...] + merge
    out = jnp.dot(out, f_W_ref[...],
                  preferred_element_type=jnp.float32) + f_b_ref[...]
    out_ref[...] = residual_ref[...] + out


def kernel(query, key_t, query_pos, key_pos, q_ln_g, q_ln_b, q_W, q_b,
           k_ln_g, k_ln_b, k_W, k_b, p_W, p_b, f_W, f_b, alpha):
    q2 = query[:, 0, :]
    qp2 = query_pos[:, 0, :]
    k2 = key_t[:, 0, :]
    kp2 = key_pos[:, 0, :]

    q4n, qp, k4n, v = pl.pallas_call(
        _stage_a_kernel,
        out_shape=[
            jax.ShapeDtypeStruct((Q, D), jnp.float32),
            jax.ShapeDtypeStruct((Q, D), jnp.float32),
            jax.ShapeDtypeStruct((K, D), jnp.float32),
            jax.ShapeDtypeStruct((K, D), jnp.float32),
        ],
    )(q2, qp2, k2, kp2, q_ln_g, q_ln_b, q_W, q_b, k_ln_g, k_ln_b, k_W, k_b)

    q4h = jnp.transpose(q4n.reshape(Q, H, HD), (1, 0, 2))
    k4h = jnp.transpose(k4n.reshape(K, H, HD), (1, 0, 2))
    vh = jnp.transpose(v.reshape(K, H, HD), (1, 0, 2))

    merge_h = pl.pallas_call(
        _stage_b_kernel,
        grid=(H,),
        in_specs=[
            pl.BlockSpec((1, Q, HD), lambda h: (h, 0, 0)),
            pl.BlockSpec((1, K, HD), lambda h: (h, 0, 0)),
            pl.BlockSpec((1, K, HD), lambda h: (h, 0, 0)),
        ],
        out_specs=pl.BlockSpec((1, Q, HD), lambda h: (h, 0, 0)),
        out_shape=jax.ShapeDtypeStruct((H, Q, HD), jnp.float32),
    )(q4h, k4h, vh)

    merge = jnp.transpose(merge_h, (1, 0, 2)).reshape(Q, D)

    out = pl.pallas_call(
        _stage_c_kernel,
        out_shape=jax.ShapeDtypeStruct((Q, D), jnp.float32),
    )(merge, qp, q2, p_W, p_b, f_W, f_b, alpha[0])

    return out[:, None, :]


# speedup vs baseline: 77.7774x; 77.7774x over previous
import math
import functools

import jax
import jax.numpy as jnp
from jax.experimental import pallas as pl
from jax.experimental.pallas import tpu as pltpu

D = 768
H = 12
HD = 64
Q = 256
K = 4096
KTU = max(32, min(int(math.ceil(0.1 * K)), K))
SCALE = HD ** -0.5

_ONE_BITS = 0x3F800000


def _seg_matrix():
    lane = jax.lax.broadcasted_iota(jnp.int32, (D, H), 0)
    head = jax.lax.broadcasted_iota(jnp.int32, (D, H), 1)
    return (lane // HD == head).astype(jnp.float32)


def _ln_norm_proj(x, g, b, W, bias):
    m = jnp.mean(x, axis=-1, keepdims=True)
    v = jnp.mean((x - m) ** 2, axis=-1, keepdims=True)
    xn = (x - m) * jax.lax.rsqrt(v + 1e-5) * g + b
    return jnp.dot(xn, W, preferred_element_type=jnp.float32) + bias


def _headwise_l2norm(x, seg):
    ssum = jnp.dot(x * x, seg, preferred_element_type=jnp.float32)
    nrm = jnp.maximum(jnp.sqrt(ssum), 1e-12)
    inv_full = jnp.dot(1.0 / nrm, seg.T, preferred_element_type=jnp.float32)
    return x * inv_full


def _stage_aq_kernel(query_ref, qpos_ref,
                     q_ln_g_ref, q_ln_b_ref, q_W_ref, q_b_ref,
                     q4n_ref, qp_ref):
    seg = _seg_matrix()
    q = query_ref[...] + qpos_ref[...]
    qp = _ln_norm_proj(q, q_ln_g_ref[...], q_ln_b_ref[...],
                       q_W_ref[...], q_b_ref[...])
    qp_ref[...] = qp
    q4n_ref[...] = _headwise_l2norm(qp, seg)


def _stage_ak_kernel(key_ref, kpos_ref,
                     k_ln_g_ref, k_ln_b_ref, k_W_ref, k_b_ref,
                     k4n_ref, v_ref):
    seg = _seg_matrix()
    kk = key_ref[...] + kpos_ref[...]
    v_ref[...] = kk
    kp = _ln_norm_proj(kk, k_ln_g_ref[...], k_ln_b_ref[...],
                       k_W_ref[...], k_b_ref[...])
    k4n_ref[...] = _headwise_l2norm(kp, seg)


def _encode(x):
    i = jax.lax.bitcast_convert_type(x, jnp.int32)
    return i ^ jax.lax.shift_right_logical(
        jax.lax.shift_right_arithmetic(i, 31), 1)


def _stage_b_kernel(q_ref, k_ref, v_ref, out_ref):
    q = q_ref[0]
    k = k_ref[0]
    v = v_ref[0]
    sim = jax.lax.dot_general(q, k, (((1,), (1,)), ((), ())),
                              preferred_element_type=jnp.float32)
    enc = _encode(sim)

    def body(_, carry):
        lo, hi = carry
        mid = lo + jax.lax.shift_right_arithmetic(hi - lo, 1)
        cnt = jnp.sum((enc >= mid).astype(jnp.int32), axis=-1, keepdims=True)
        ge = cnt >= KTU
        return jnp.where(ge, mid, lo), jnp.where(ge, hi, mid)

    lo0 = jnp.full((Q, 1), -(_ONE_BITS + 1), jnp.int32)
    hi0 = jnp.full((Q, 1), _ONE_BITS + 1, jnp.int32)
    lo, _ = jax.lax.fori_loop(0, 31, body, (lo0, hi0))

    p = jnp.where(enc >= lo, jnp.exp(sim * SCALE), 0.0)
    attn = p / jnp.sum(p, axis=-1, keepdims=True)
    out_ref[0] = jnp.dot(attn, v, preferred_element_type=jnp.float32)


def _stage_c_kernel(merge_ref, qp_ref, residual_ref,
                    p_W_ref, p_b_ref, f_W_ref, f_b_ref, alpha_ref, out_ref):
    merge = merge_ref[...]
    inter = jnp.dot(merge * qp_ref[...], p_W_ref[...],
                    preferred_element_type=jnp.float32) + p_b_ref[...]
    n2 = jnp.sum(inter * inter, axis=0, keepdims=True)
    nrm = jnp.maximum(jnp.sqrt(n2), 1e-12)
    out = inter / nrm * alpha_ref[...] + merge
    out = jnp.dot(out, f_W_ref[...],
                  preferred_element_type=jnp.float32) + f_b_ref[...]
    out_ref[...] = residual_ref[...] + out


def kernel(query, key_t, query_pos, key_pos, q_ln_g, q_ln_b, q_W, q_b,
           k_ln_g, k_ln_b, k_W, k_b, p_W, p_b, f_W, f_b, alpha):
    q2 = query[:, 0, :]
    qp2 = query_pos[:, 0, :]
    k2 = key_t[:, 0, :]
    kp2 = key_pos[:, 0, :]

    q4n, qp = pl.pallas_call(
        _stage_aq_kernel,
        out_shape=[
            jax.ShapeDtypeStruct((Q, D), jnp.float32),
            jax.ShapeDtypeStruct((Q, D), jnp.float32),
        ],
    )(q2, qp2, q_ln_g, q_ln_b, q_W, q_b)

    KB = 1024
    k4n, v = pl.pallas_call(
        _stage_ak_kernel,
        grid=(K // KB,),
        in_specs=[
            pl.BlockSpec((KB, D), lambda i: (i, 0)),
            pl.BlockSpec((KB, D), lambda i: (i, 0)),
            pl.BlockSpec((D,), lambda i: (0,)),
            pl.BlockSpec((D,), lambda i: (0,)),
            pl.BlockSpec((D, D), lambda i: (0, 0)),
            pl.BlockSpec((D,), lambda i: (0,)),
        ],
        out_specs=[
            pl.BlockSpec((KB, D), lambda i: (i, 0)),
            pl.BlockSpec((KB, D), lambda i: (i, 0)),
        ],
        out_shape=[
            jax.ShapeDtypeStruct((K, D), jnp.float32),
            jax.ShapeDtypeStruct((K, D), jnp.float32),
        ],
    )(k2, kp2, k_ln_g, k_ln_b, k_W, k_b)

    q4h = jnp.transpose(q4n.reshape(Q, H, HD), (1, 0, 2))
    k4h = jnp.transpose(k4n.reshape(K, H, HD), (1, 0, 2))
    vh = jnp.transpose(v.reshape(K, H, HD), (1, 0, 2))

    merge_h = pl.pallas_call(
        _stage_b_kernel,
        grid=(H,),
        in_specs=[
            pl.BlockSpec((1, Q, HD), lambda h: (h, 0, 0)),
            pl.BlockSpec((1, K, HD), lambda h: (h, 0, 0)),
            pl.BlockSpec((1, K, HD), lambda h: (h, 0, 0)),
        ],
        out_specs=pl.BlockSpec((1, Q, HD), lambda h: (h, 0, 0)),
        out_shape=jax.ShapeDtypeStruct((H, Q, HD), jnp.float32),
    )(q4h, k4h, vh)

    merge = jnp.transpose(merge_h, (1, 0, 2)).reshape(Q, D)

    out = pl.pallas_call(
        _stage_c_kernel,
        out_shape=jax.ShapeDtypeStruct((Q, D), jnp.float32),
    )(merge, qp, q2, p_W, p_b, f_W, f_b, alpha[0])

    return out[:, None, :]


# speedup vs baseline: 111.7991x; 1.4374x over previous
import math
import functools

import jax
import jax.numpy as jnp
from jax.experimental import pallas as pl
from jax.experimental.pallas import tpu as pltpu

D = 768
H = 12
HD = 64
Q = 256
K = 4096
KTU = max(32, min(int(math.ceil(0.1 * K)), K))
SCALE = HD ** -0.5

_ONE_BITS = 0x3F800000


def _seg_matrix():
    lane = jax.lax.broadcasted_iota(jnp.int32, (D, H), 0)
    head = jax.lax.broadcasted_iota(jnp.int32, (D, H), 1)
    return (lane // HD == head).astype(jnp.float32)


def _ln_norm_proj(x, g, b, W, bias):
    m = jnp.mean(x, axis=-1, keepdims=True)
    v = jnp.mean((x - m) ** 2, axis=-1, keepdims=True)
    xn = (x - m) * jax.lax.rsqrt(v + 1e-5) * g + b
    return jnp.dot(xn, W, preferred_element_type=jnp.float32) + bias


def _headwise_l2norm(x, seg):
    ssum = jnp.dot(x * x, seg, preferred_element_type=jnp.float32)
    nrm = jnp.maximum(jnp.sqrt(ssum), 1e-12)
    inv_full = jnp.dot(1.0 / nrm, seg.T, preferred_element_type=jnp.float32)
    return x * inv_full


def _stage_aq_kernel(query_ref, qpos_ref,
                     q_ln_g_ref, q_ln_b_ref, q_W_ref, q_b_ref,
                     q4n_ref, qp_ref):
    seg = _seg_matrix()
    q = query_ref[...] + qpos_ref[...]
    qp = _ln_norm_proj(q, q_ln_g_ref[...], q_ln_b_ref[...],
                       q_W_ref[...], q_b_ref[...])
    qp_ref[...] = qp
    q4n_ref[...] = _headwise_l2norm(qp, seg)


def _stage_ak_kernel(key_ref, kpos_ref,
                     k_ln_g_ref, k_ln_b_ref, k_W_ref, k_b_ref,
                     k4n_ref, v_ref):
    seg = _seg_matrix()
    kk = key_ref[...] + kpos_ref[...]
    v_ref[...] = kk
    kp = _ln_norm_proj(kk, k_ln_g_ref[...], k_ln_b_ref[...],
                       k_W_ref[...], k_b_ref[...])
    k4n_ref[...] = _headwise_l2norm(kp, seg)


def _encode(x):
    i = jax.lax.bitcast_convert_type(x, jnp.int32)
    return i ^ jax.lax.shift_right_logical(
        jax.lax.shift_right_arithmetic(i, 31), 1)


N_ITERS = 24


def _attend(q, k, v):
    sim = jax.lax.dot_general(q, k, (((1,), (1,)), ((), ())),
                              preferred_element_type=jnp.float32)
    enc = _encode(sim)

    def body(_, carry):
        lo, hi = carry
        mid = lo + jax.lax.shift_right_arithmetic(hi - lo, 1)
        cnt = jnp.sum((enc >= mid).astype(jnp.int32), axis=-1, keepdims=True)
        ge = cnt >= KTU
        return jnp.where(ge, mid, lo), jnp.where(ge, hi, mid)

    lo0 = jnp.full((Q, 1), -(_ONE_BITS + 1), jnp.int32)
    hi0 = jnp.full((Q, 1), _ONE_BITS + 1, jnp.int32)
    lo, _ = jax.lax.fori_loop(0, N_ITERS, body, (lo0, hi0))

    p = jnp.where(enc >= lo, jnp.exp(sim * SCALE), 0.0)
    attn = p / jnp.sum(p, axis=-1, keepdims=True)
    return jnp.dot(attn, v, preferred_element_type=jnp.float32)


def _stage_b_kernel(q_ref, k_ref, v_ref, out_ref):
    for h in range(2):
        sl = slice(h * HD, (h + 1) * HD)
        out_ref[:, sl] = _attend(q_ref[:, sl], k_ref[:, sl], v_ref[:, sl])


def _stage_c_kernel(merge_ref, qp_ref, residual_ref,
                    p_W_ref, p_b_ref, f_W_ref, f_b_ref, alpha_ref, out_ref):
    merge = merge_ref[...]
    inter = jnp.dot(merge * qp_ref[...], p_W_ref[...],
                    preferred_element_type=jnp.float32) + p_b_ref[...]
    n2 = jnp.sum(inter * inter, axis=0, keepdims=True)
    nrm = jnp.maximum(jnp.sqrt(n2), 1e-12)
    out = inter / nrm * alpha_ref[...] + merge
    out = jnp.dot(out, f_W_ref[...],
                  preferred_element_type=jnp.float32) + f_b_ref[...]
    out_ref[...] = residual_ref[...] + out


def kernel(query, key_t, query_pos, key_pos, q_ln_g, q_ln_b, q_W, q_b,
           k_ln_g, k_ln_b, k_W, k_b, p_W, p_b, f_W, f_b, alpha):
    q2 = query[:, 0, :]
    qp2 = query_pos[:, 0, :]
    k2 = key_t[:, 0, :]
    kp2 = key_pos[:, 0, :]

    q4n, qp = pl.pallas_call(
        _stage_aq_kernel,
        out_shape=[
            jax.ShapeDtypeStruct((Q, D), jnp.float32),
            jax.ShapeDtypeStruct((Q, D), jnp.float32),
        ],
    )(q2, qp2, q_ln_g, q_ln_b, q_W, q_b)

    KB = 1024
    k4n, v = pl.pallas_call(
        _stage_ak_kernel,
        grid=(K // KB,),
        in_specs=[
            pl.BlockSpec((KB, D), lambda i: (i, 0)),
            pl.BlockSpec((KB, D), lambda i: (i, 0)),
            pl.BlockSpec((D,), lambda i: (0,)),
            pl.BlockSpec((D,), lambda i: (0,)),
            pl.BlockSpec((D, D), lambda i: (0, 0)),
            pl.BlockSpec((D,), lambda i: (0,)),
        ],
        out_specs=[
            pl.BlockSpec((KB, D), lambda i: (i, 0)),
            pl.BlockSpec((KB, D), lambda i: (i, 0)),
        ],
        out_shape=[
            jax.ShapeDtypeStruct((K, D), jnp.float32),
            jax.ShapeDtypeStruct((K, D), jnp.float32),
        ],
    )(k2, kp2, k_ln_g, k_ln_b, k_W, k_b)

    merge = pl.pallas_call(
        _stage_b_kernel,
        grid=(H // 2,),
        in_specs=[
            pl.BlockSpec((Q, 2 * HD), lambda h: (0, h)),
            pl.BlockSpec((K, 2 * HD), lambda h: (0, h)),
            pl.BlockSpec((K, 2 * HD), lambda h: (0, h)),
        ],
        out_specs=pl.BlockSpec((Q, 2 * HD), lambda h: (0, h)),
        out_shape=jax.ShapeDtypeStruct((Q, D), jnp.float32),
    )(q4n, k4n, v)

    out = pl.pallas_call(
        _stage_c_kernel,
        out_shape=jax.ShapeDtypeStruct((Q, D), jnp.float32),
    )(merge, qp, q2, p_W, p_b, f_W, f_b, alpha[0])

    return out[:, None, :]
